# baseline (speedup 1.0000x reference)
import functools

import jax
import jax.numpy as jnp
from jax import lax
from jax.experimental import pallas as pl
from jax.experimental.pallas import tpu as pltpu
from jax.experimental.pallas import tpu_sc as plsc

EMB = 64
SEQ = 50

_info = plsc.get_sparse_core_info()
_NC = _info.num_cores
_NS = _info.num_subcores
_NW = _NC * _NS

CHUNK = 128
NBUF = 4


def _sc_gather_t(idx2, table, n_cols):
    n_blocks = idx2.shape[0]
    assert n_blocks % _NW == 0
    blocks_per_w = n_blocks // _NW
    assert blocks_per_w % NBUF == 0
    n_outer = blocks_per_w // NBUF
    cpr = n_cols // CHUNK

    mesh = plsc.VectorSubcoreMesh(core_axis_name="c", subcore_axis_name="s")

    scratch = (
        [pltpu.VMEM((blocks_per_w, CHUNK), jnp.int32)]
        + [pltpu.VMEM((CHUNK, EMB), jnp.float32) for _ in range(NBUF)]
        + [pltpu.VMEM((EMB, CHUNK + 1), jnp.float32) for _ in range(NBUF)]
        + [pltpu.SemaphoreType.DMA for _ in range(2 * NBUF)]
    )

    @functools.partial(
        pl.kernel,
        mesh=mesh,
        out_type=jax.ShapeDtypeStruct((SEQ, EMB, n_cols), jnp.float32),
        compiler_params=pltpu.CompilerParams(
            use_tc_tiling_on_sc=False, needs_layout_passes=False
        ),
        scratch_types=scratch,
    )
    def k(idx_hbm, table_hbm, out_hbm, idx_v, *bufs_and_sems):
        m_bufs = bufs_and_sems[:NBUF]
        t_bufs = bufs_and_sems[NBUF : 2 * NBUF]
        gsem = bufs_and_sems[2 * NBUF : 3 * NBUF]
        osem = bufs_and_sems[3 * NBUF : 4 * NBUF]

        wid = lax.axis_index("s") * _NC + lax.axis_index("c")
        base_blk = wid * blocks_per_w
        pltpu.sync_copy(idx_hbm.at[pl.ds(base_blk, blocks_per_w)], idx_v)

        lanes = jnp.arange(16, dtype=jnp.int32)
        jvecs = [lanes + 16 * jc for jc in range(CHUNK // 16)]

        def gather_start(t, b):
            pltpu.async_copy(table_hbm.at[idx_v.at[t]], m_bufs[b], gsem[b])

        def gather_wait(t, b):
            pltpu.make_async_copy(
                table_hbm.at[idx_v.at[t]], m_bufs[b], gsem[b]
            ).wait()

        def out_slice(t, b):
            blk = base_blk + t
            s = blk // cpr
            c = blk % cpr
            return out_hbm.at[s, :, pl.ds(c * CHUNK, CHUNK)]

        def write_start(t, b):
            pltpu.async_copy(
                t_bufs[b].at[:, pl.ds(0, CHUNK)], out_slice(t, b), osem[b]
            )

        def write_wait(t, b):
            pltpu.make_async_copy(
                t_bufs[b].at[:, pl.ds(0, CHUNK)], out_slice(t, b), osem[b]
            ).wait()

        evecs = [lanes + 16 * ec for ec in range(EMB // 16)]

        def transpose(b):
            m = m_bufs[b]
            mt = t_bufs[b]

            def jrow(j2, carry):
                for ju in range(4):
                    j = j2 * 4 + ju
                    js = jnp.full((16,), 0, jnp.int32) + j
                    for ec in range(EMB // 16):
                        v = m[j, pl.ds(ec * 16, 16)]
                        plsc.store_scatter(mt, [evecs[ec], js], v)
                return carry

            lax.fori_loop(0, CHUNK // 4, jrow, 0)

        for b in range(NBUF):
            gather_start(b, b)

        def outer(o, carry):
            for b in range(NBUF):
                t = o * NBUF + b

                gather_wait(t, b)

                @pl.when(o > 0)
                def _retire():
                    write_wait(t - NBUF, b)

                transpose(b)
                write_start(t, b)

                @pl.when(o < n_outer - 1)
                def _refill():
                    gather_start(t + NBUF, b)

            return carry

        lax.fori_loop(0, n_outer, outer, 0)

        for b in range(NBUF):
            write_wait((n_outer - 1) * NBUF + b, b)

    return k(idx2, table)


def kernel(indices, table):
    batch, seq = indices.shape
    idx2 = indices.T.reshape(seq * batch // CHUNK, CHUNK).astype(jnp.int32)
    out_t = _sc_gather_t(idx2, table, batch)
    return out_t.transpose(2, 0, 1)


# speedup vs baseline: 1.1115x; 1.1115x over previous
import functools

import jax
import jax.numpy as jnp
from jax import lax
from jax.experimental import pallas as pl
from jax.experimental.pallas import tpu as pltpu
from jax.experimental.pallas import tpu_sc as plsc

EMB = 64
SEQ = 50

_info = plsc.get_sparse_core_info()
_NC = _info.num_cores
_NS = _info.num_subcores
_NW = _NC * _NS

CHUNK = 128
NBUF = 4


def _sc_gather_t(idx2, table, n_cols):
    n_blocks = idx2.shape[0]
    assert n_blocks % _NW == 0
    blocks_per_w = n_blocks // _NW
    assert blocks_per_w % NBUF == 0
    n_outer = blocks_per_w // NBUF
    cpr = n_cols // CHUNK

    mesh = plsc.VectorSubcoreMesh(core_axis_name="c", subcore_axis_name="s")

    scratch = (
        [pltpu.VMEM((blocks_per_w, CHUNK), jnp.int32)]
        + [pltpu.VMEM((CHUNK, EMB), jnp.float32) for _ in range(NBUF)]
        + [pltpu.VMEM((EMB, CHUNK + 1), jnp.float32) for _ in range(NBUF)]
        + [pltpu.SemaphoreType.DMA for _ in range(2 * NBUF)]
    )

    @functools.partial(
        pl.kernel,
        mesh=mesh,
        out_type=jax.ShapeDtypeStruct((SEQ, EMB, n_cols), jnp.float32),
        compiler_params=pltpu.CompilerParams(
            use_tc_tiling_on_sc=False, needs_layout_passes=False
        ),
        scratch_types=scratch,
    )
    def k(idx_hbm, table_hbm, out_hbm, idx_v, *bufs_and_sems):
        m_bufs = bufs_and_sems[:NBUF]
        t_bufs = bufs_and_sems[NBUF : 2 * NBUF]
        gsem = bufs_and_sems[2 * NBUF : 3 * NBUF]
        osem = bufs_and_sems[3 * NBUF : 4 * NBUF]

        wid = lax.axis_index("s") * _NC + lax.axis_index("c")
        base_blk = wid * blocks_per_w
        pltpu.sync_copy(idx_hbm.at[pl.ds(base_blk, blocks_per_w)], idx_v)

        lanes = jnp.arange(16, dtype=jnp.int32)
        jvecs = [lanes + 16 * jc for jc in range(CHUNK // 16)]

        def gather_start(t, b):
            pltpu.async_copy(table_hbm.at[idx_v.at[t]], m_bufs[b], gsem[b])

        def gather_wait(t, b):
            pltpu.make_async_copy(
                table_hbm.at[idx_v.at[t]], m_bufs[b], gsem[b]
            ).wait()

        def out_slice(t, b):
            blk = base_blk + t
            s = blk // cpr
            c = blk % cpr
            return out_hbm.at[s, :, pl.ds(c * CHUNK, CHUNK)]

        def write_start(t, b):
            pltpu.async_copy(
                t_bufs[b].at[:, pl.ds(0, CHUNK)], out_slice(t, b), osem[b]
            )

        def write_wait(t, b):
            pltpu.make_async_copy(
                t_bufs[b].at[:, pl.ds(0, CHUNK)], out_slice(t, b), osem[b]
            ).wait()

        evecs = [lanes + 16 * ec for ec in range(EMB // 16)]

        def transpose(b):
            m = m_bufs[b]
            mt = t_bufs[b]

            def jrow(j2, carry):
                jss = []
                vals = []
                for ju in range(4):
                    j = j2 * 4 + ju
                    jss.append(jnp.full((16,), 0, jnp.int32) + j)
                    for ec in range(EMB // 16):
                        vals.append(m[j, pl.ds(ec * 16, 16)])
                k = 0
                for ju in range(4):
                    for ec in range(EMB // 16):
                        plsc.store_scatter(mt, [evecs[ec], jss[ju]], vals[k])
                        k += 1
                return carry

            lax.fori_loop(0, CHUNK // 4, jrow, 0)

        for b in range(NBUF):
            gather_start(b, b)

        def outer(o, carry):
            for b in range(NBUF):
                t = o * NBUF + b

                gather_wait(t, b)

                @pl.when(o > 0)
                def _retire():
                    write_wait(t - NBUF, b)

                transpose(b)
                write_start(t, b)

                @pl.when(o < n_outer - 1)
                def _refill():
                    gather_start(t + NBUF, b)

            return carry

        lax.fori_loop(0, n_outer, outer, 0)

        for b in range(NBUF):
            write_wait((n_outer - 1) * NBUF + b, b)

    return k(idx2, table)


def kernel(indices, table):
    batch, seq = indices.shape
    idx2 = indices.T.reshape(seq * batch // CHUNK, CHUNK).astype(jnp.int32)
    out_t = _sc_gather_t(idx2, table, batch)
    return out_t.transpose(2, 0, 1)


# speedup vs baseline: 1.1423x; 1.0277x over previous
import functools

import jax
import jax.numpy as jnp
from jax import lax
from jax.experimental import pallas as pl
from jax.experimental.pallas import tpu as pltpu
from jax.experimental.pallas import tpu_sc as plsc

EMB = 64
SEQ = 50

_info = plsc.get_sparse_core_info()
_NC = _info.num_cores
_NS = _info.num_subcores
_NW = _NC * _NS

CHUNK = 128
NBUF = 2
TPAD = 128


def _sc_gather_t(idx2, table, n_cols):
    n_blocks = idx2.shape[0]
    assert n_blocks % _NW == 0
    blocks_per_w = n_blocks // _NW
    assert blocks_per_w % NBUF == 0
    n_outer = blocks_per_w // NBUF
    cpr = n_cols // CHUNK

    mesh = plsc.VectorSubcoreMesh(core_axis_name="c", subcore_axis_name="s")

    scratch = (
        [pltpu.VMEM((blocks_per_w, CHUNK), jnp.int32)]
        + [pltpu.VMEM((CHUNK, TPAD), jnp.float32) for _ in range(NBUF)]
        + [pltpu.VMEM((EMB, CHUNK + 1), jnp.float32) for _ in range(NBUF)]
        + [pltpu.SemaphoreType.DMA for _ in range(2 * NBUF)]
    )

    @functools.partial(
        pl.kernel,
        mesh=mesh,
        out_type=jax.ShapeDtypeStruct((SEQ, EMB, n_cols), jnp.float32),
        compiler_params=pltpu.CompilerParams(
            use_tc_tiling_on_sc=False, needs_layout_passes=False
        ),
        scratch_types=scratch,
    )
    def k(idx_hbm, table_hbm, out_hbm, idx_v, *bufs_and_sems):
        m_bufs = bufs_and_sems[:NBUF]
        t_bufs = bufs_and_sems[NBUF : 2 * NBUF]
        gsem = bufs_and_sems[2 * NBUF : 3 * NBUF]
        osem = bufs_and_sems[3 * NBUF : 4 * NBUF]

        wid = lax.axis_index("s") * _NC + lax.axis_index("c")
        base_blk = wid * blocks_per_w
        pltpu.sync_copy(idx_hbm.at[pl.ds(base_blk, blocks_per_w)], idx_v)

        lanes = jnp.arange(16, dtype=jnp.int32)
        jvecs = [lanes + 16 * jc for jc in range(CHUNK // 16)]

        def gather_start(t, b):
            pltpu.async_copy(table_hbm.at[idx_v.at[t]], m_bufs[b], gsem[b])

        def gather_wait(t, b):
            pltpu.make_async_copy(
                table_hbm.at[idx_v.at[t]], m_bufs[b], gsem[b]
            ).wait()

        def out_slice(t, b):
            blk = base_blk + t
            s = blk // cpr
            c = blk % cpr
            return out_hbm.at[s, :, pl.ds(c * CHUNK, CHUNK)]

        def write_start(t, b):
            pltpu.async_copy(
                t_bufs[b].at[:, pl.ds(0, CHUNK)], out_slice(t, b), osem[b]
            )

        def write_wait(t, b):
            pltpu.make_async_copy(
                t_bufs[b].at[:, pl.ds(0, CHUNK)], out_slice(t, b), osem[b]
            ).wait()

        evecs = [lanes + 16 * ec for ec in range(EMB // 16)]

        def transpose(b):
            m = m_bufs[b]
            mt = t_bufs[b]

            def jrow(j2, carry):
                jss = []
                vals = []
                for ju in range(4):
                    j = j2 * 4 + ju
                    jss.append(jnp.full((16,), 0, jnp.int32) + j)
                    for ec in range(EMB // 16):
                        vals.append(m[j, pl.ds(ec * 16, 16)])
                k = 0
                for ju in range(4):
                    for ec in range(EMB // 16):
                        plsc.store_scatter(mt, [evecs[ec], jss[ju]], vals[k])
                        k += 1
                return carry

            lax.fori_loop(0, CHUNK // 4, jrow, 0)

        for b in range(NBUF):
            gather_start(b, b)

        def outer(o, carry):
            for b in range(NBUF):
                t = o * NBUF + b

                gather_wait(t, b)

                @pl.when(o > 0)
                def _retire():
                    write_wait(t - NBUF, b)

                transpose(b)
                write_start(t, b)

                @pl.when(o < n_outer - 1)
                def _refill():
                    gather_start(t + NBUF, b)

            return carry

        lax.fori_loop(0, n_outer, outer, 0)

        for b in range(NBUF):
            write_wait((n_outer - 1) * NBUF + b, b)

    return k(idx2, table)


def kernel(indices, table):
    batch, seq = indices.shape
    idx2 = indices.T.reshape(seq * batch // CHUNK, CHUNK).astype(jnp.int32)
    table = jnp.pad(table, ((0, 0), (0, TPAD - EMB)))
    out_t = _sc_gather_t(idx2, table, batch)
    return out_t.transpose(2, 0, 1)


# speedup vs baseline: 1.1818x; 1.0346x over previous
import functools

import jax
import jax.numpy as jnp
from jax import lax
from jax.experimental import pallas as pl
from jax.experimental.pallas import tpu as pltpu
from jax.experimental.pallas import tpu_sc as plsc

EMB = 64
SEQ = 50

_info = plsc.get_sparse_core_info()
_NC = _info.num_cores
_NS = _info.num_subcores
_NW = _NC * _NS

CHUNK = 128
NBUF = 2
TPAD = 128


def _sc_gather_t(idx2, table, n_cols):
    n_blocks = idx2.shape[0]
    assert n_blocks % _NW == 0
    blocks_per_w = n_blocks // _NW
    assert blocks_per_w % NBUF == 0
    n_outer = blocks_per_w // NBUF
    cpr = n_cols // CHUNK

    mesh = plsc.VectorSubcoreMesh(core_axis_name="c", subcore_axis_name="s")

    scratch = (
        [pltpu.VMEM((blocks_per_w, CHUNK), jnp.int32)]
        + [pltpu.VMEM((CHUNK, EMB), jnp.float32) for _ in range(NBUF)]
        + [pltpu.VMEM((EMB, CHUNK + 1), jnp.float32) for _ in range(NBUF)]
        + [pltpu.SemaphoreType.DMA for _ in range(2 * NBUF)]
    )

    @functools.partial(
        pl.kernel,
        mesh=mesh,
        out_type=jax.ShapeDtypeStruct((SEQ, EMB, n_cols), jnp.float32),
        compiler_params=pltpu.CompilerParams(
            use_tc_tiling_on_sc=False, needs_layout_passes=False
        ),
        scratch_types=scratch,
    )
    def k(idx_hbm, table_hbm, out_hbm, idx_v, *bufs_and_sems):
        m_bufs = bufs_and_sems[:NBUF]
        t_bufs = bufs_and_sems[NBUF : 2 * NBUF]
        gsem = bufs_and_sems[2 * NBUF : 3 * NBUF]
        osem = bufs_and_sems[3 * NBUF : 4 * NBUF]

        wid = lax.axis_index("s") * _NC + lax.axis_index("c")
        base_blk = wid * blocks_per_w
        pltpu.sync_copy(idx_hbm.at[pl.ds(base_blk, blocks_per_w)], idx_v)

        lanes = jnp.arange(16, dtype=jnp.int32)
        jvecs = [lanes + 16 * jc for jc in range(CHUNK // 16)]

        def gather_start(t, b):
            pltpu.async_copy(table_hbm.at[idx_v.at[t]], m_bufs[b], gsem[b])

        def gather_wait(t, b):
            pltpu.make_async_copy(
                table_hbm.at[idx_v.at[t]], m_bufs[b], gsem[b]
            ).wait()

        def out_slice(t, b):
            blk = base_blk + t
            s = blk // cpr
            c = blk % cpr
            return out_hbm.at[s, :, pl.ds(c * CHUNK, CHUNK)]

        def write_start(t, b):
            pltpu.async_copy(
                t_bufs[b].at[:, pl.ds(0, CHUNK)], out_slice(t, b), osem[b]
            )

        def write_wait(t, b):
            pltpu.make_async_copy(
                t_bufs[b].at[:, pl.ds(0, CHUNK)], out_slice(t, b), osem[b]
            ).wait()

        evecs = [lanes + 16 * ec for ec in range(EMB // 16)]

        def transpose(b):
            m = m_bufs[b]
            mt = t_bufs[b]

            def jrow(j2, carry):
                jss = []
                vals = []
                for ju in range(4):
                    j = j2 * 4 + ju
                    jss.append(jnp.full((16,), 0, jnp.int32) + j)
                    for ec in range(EMB // 16):
                        vals.append(m[j, pl.ds(ec * 16, 16)])
                k = 0
                for ju in range(4):
                    for ec in range(EMB // 16):
                        plsc.store_scatter(mt, [evecs[ec], jss[ju]], vals[k])
                        k += 1
                return carry

            lax.fori_loop(0, CHUNK // 4, jrow, 0)

        for b in range(NBUF):
            gather_start(b, b)

        def outer(o, carry):
            for b in range(NBUF):
                t = o * NBUF + b

                gather_wait(t, b)

                @pl.when(o > 0)
                def _retire():
                    write_wait(t - NBUF, b)

                transpose(b)
                write_start(t, b)

                @pl.when(o < n_outer - 1)
                def _refill():
                    gather_start(t + NBUF, b)

            return carry

        lax.fori_loop(0, n_outer, outer, 0)

        for b in range(NBUF):
            write_wait((n_outer - 1) * NBUF + b, b)

    return k(idx2, table)


def kernel(indices, table):
    batch, seq = indices.shape
    idx2 = indices.T.reshape(seq * batch // CHUNK, CHUNK).astype(jnp.int32)
    table = jnp.pad(table, ((0, 0), (0, TPAD - EMB)))
    table = table.reshape(table.shape[0] * (TPAD // EMB), EMB)
    idx2 = idx2 * (TPAD // EMB)
    out_t = _sc_gather_t(idx2, table, batch)
    return out_t.transpose(2, 0, 1)


# speedup vs baseline: 1.5128x; 1.2800x over previous
import functools

import jax
import jax.numpy as jnp
from jax import lax
from jax.experimental import pallas as pl
from jax.experimental.pallas import tpu as pltpu
from jax.experimental.pallas import tpu_sc as plsc

EMB = 64
SEQ = 50

_info = plsc.get_sparse_core_info()
_NC = _info.num_cores
_NS = _info.num_subcores
_NW = _NC * _NS

CHUNK = 128
NBUF = 2
TPAD = 128


def _sc_gather_t(idx2, table, n_cols):
    n_blocks = idx2.shape[0]
    assert n_blocks % _NW == 0
    blocks_per_w = n_blocks // _NW
    assert blocks_per_w % NBUF == 0
    n_outer = blocks_per_w // NBUF
    cpr = n_cols // CHUNK

    mesh = plsc.VectorSubcoreMesh(core_axis_name="c", subcore_axis_name="s")

    scratch = (
        [pltpu.VMEM((blocks_per_w, CHUNK), jnp.int32)]
        + [pltpu.VMEM((CHUNK, EMB), jnp.float32) for _ in range(NBUF)]
        + [pltpu.VMEM((EMB, CHUNK + 1), jnp.float32) for _ in range(NBUF)]
        + [pltpu.SemaphoreType.DMA for _ in range(2 * NBUF)]
    )

    @functools.partial(
        pl.kernel,
        mesh=mesh,
        out_type=jax.ShapeDtypeStruct(
            (SEQ * (EMB // 8) * cpr, 8, CHUNK), jnp.float32
        ),
        compiler_params=pltpu.CompilerParams(
            use_tc_tiling_on_sc=False, needs_layout_passes=False
        ),
        scratch_types=scratch,
    )
    def k(idx_hbm, table_hbm, out_hbm, idx_v, *bufs_and_sems):
        m_bufs = bufs_and_sems[:NBUF]
        t_bufs = bufs_and_sems[NBUF : 2 * NBUF]
        gsem = bufs_and_sems[2 * NBUF : 3 * NBUF]
        osem = bufs_and_sems[3 * NBUF : 4 * NBUF]

        wid = lax.axis_index("s") * _NC + lax.axis_index("c")
        base_blk = wid * blocks_per_w
        pltpu.sync_copy(idx_hbm.at[pl.ds(base_blk, blocks_per_w)], idx_v)

        lanes = jnp.arange(16, dtype=jnp.int32)
        jvecs = [lanes + 16 * jc for jc in range(CHUNK // 16)]

        def gather_start(t, b):
            pltpu.async_copy(table_hbm.at[idx_v.at[t]], m_bufs[b], gsem[b])

        def gather_wait(t, b):
            pltpu.make_async_copy(
                table_hbm.at[idx_v.at[t]], m_bufs[b], gsem[b]
            ).wait()

        def write_start(t, b):
            blk = base_blk + t
            s = blk // cpr
            c = blk % cpr
            for eb in range(EMB // 8):
                tile = (s * (EMB // 8) + eb) * cpr + c
                pltpu.async_copy(
                    t_bufs[b].at[pl.ds(8 * eb, 8), pl.ds(0, CHUNK)],
                    out_hbm.at[tile],
                    osem[b],
                )

        def write_wait(t, b):
            blk = base_blk + t
            s = blk // cpr
            c = blk % cpr
            for eb in range(EMB // 8):
                tile = (s * (EMB // 8) + eb) * cpr + c
                pltpu.make_async_copy(
                    t_bufs[b].at[pl.ds(8 * eb, 8), pl.ds(0, CHUNK)],
                    out_hbm.at[tile],
                    osem[b],
                ).wait()

        evecs = [lanes + 16 * ec for ec in range(EMB // 16)]

        def transpose(b):
            m = m_bufs[b]
            mt = t_bufs[b]

            def jrow(j2, carry):
                jss = []
                vals = []
                for ju in range(4):
                    j = j2 * 4 + ju
                    jss.append(jnp.full((16,), 0, jnp.int32) + j)
                    for ec in range(EMB // 16):
                        vals.append(m[j, pl.ds(ec * 16, 16)])
                k = 0
                for ju in range(4):
                    for ec in range(EMB // 16):
                        plsc.store_scatter(mt, [evecs[ec], jss[ju]], vals[k])
                        k += 1
                return carry

            lax.fori_loop(0, CHUNK // 4, jrow, 0)

        for b in range(NBUF):
            gather_start(b, b)

        def outer(o, carry):
            for b in range(NBUF):
                t = o * NBUF + b

                gather_wait(t, b)

                @pl.when(o > 0)
                def _retire():
                    write_wait(t - NBUF, b)

                transpose(b)
                write_start(t, b)

                @pl.when(o < n_outer - 1)
                def _refill():
                    gather_start(t + NBUF, b)

            return carry

        lax.fori_loop(0, n_outer, outer, 0)

        for b in range(NBUF):
            write_wait((n_outer - 1) * NBUF + b, b)

    return k(idx2, table)


def kernel(indices, table):
    batch, seq = indices.shape
    idx2 = indices.T.reshape(seq * batch // CHUNK, CHUNK).astype(jnp.int32)
    table = jnp.pad(table, ((0, 0), (0, TPAD - EMB)))
    table = table.reshape(table.shape[0] * (TPAD // EMB), EMB)
    idx2 = idx2 * (TPAD // EMB)
    cpr = batch // CHUNK
    out_t = _sc_gather_t(idx2, table, batch)
    out6 = out_t.reshape(seq, EMB // 8, cpr, 8, CHUNK)
    return out6.transpose(2, 4, 0, 1, 3).reshape(batch, seq, EMB)


# speedup vs baseline: 1.5146x; 1.0012x over previous
import functools

import jax
import jax.numpy as jnp
from jax import lax
from jax.experimental import pallas as pl
from jax.experimental.pallas import tpu as pltpu
from jax.experimental.pallas import tpu_sc as plsc

EMB = 64
SEQ = 50

_info = plsc.get_sparse_core_info()
_NC = _info.num_cores
_NS = _info.num_subcores
_NW = _NC * _NS

CHUNK = 128
NBUF = 4
TPAD = 128


def _sc_gather_t(idx2, table, n_cols):
    n_blocks = idx2.shape[0]
    assert n_blocks % _NW == 0
    blocks_per_w = n_blocks // _NW
    assert blocks_per_w % NBUF == 0
    n_outer = blocks_per_w // NBUF
    cpr = n_cols // CHUNK

    mesh = plsc.VectorSubcoreMesh(core_axis_name="c", subcore_axis_name="s")

    scratch = (
        [pltpu.VMEM((blocks_per_w, CHUNK), jnp.int32)]
        + [pltpu.VMEM((CHUNK, EMB), jnp.float32) for _ in range(NBUF)]
        + [pltpu.VMEM((EMB, CHUNK + 1), jnp.float32) for _ in range(NBUF)]
        + [pltpu.SemaphoreType.DMA for _ in range(2 * NBUF)]
    )

    @functools.partial(
        pl.kernel,
        mesh=mesh,
        out_type=jax.ShapeDtypeStruct(
            (SEQ * (EMB // 8) * cpr, 8, CHUNK), jnp.float32
        ),
        compiler_params=pltpu.CompilerParams(
            use_tc_tiling_on_sc=False, needs_layout_passes=False
        ),
        scratch_types=scratch,
    )
    def k(idx_hbm, table_hbm, out_hbm, idx_v, *bufs_and_sems):
        m_bufs = bufs_and_sems[:NBUF]
        t_bufs = bufs_and_sems[NBUF : 2 * NBUF]
        gsem = bufs_and_sems[2 * NBUF : 3 * NBUF]
        osem = bufs_and_sems[3 * NBUF : 4 * NBUF]

        wid = lax.axis_index("s") * _NC + lax.axis_index("c")
        base_blk = wid * blocks_per_w
        pltpu.sync_copy(idx_hbm.at[pl.ds(base_blk, blocks_per_w)], idx_v)

        lanes = jnp.arange(16, dtype=jnp.int32)
        jvecs = [lanes + 16 * jc for jc in range(CHUNK // 16)]

        def gather_start(t, b):
            pltpu.async_copy(table_hbm.at[idx_v.at[t]], m_bufs[b], gsem[b])

        def gather_wait(t, b):
            pltpu.make_async_copy(
                table_hbm.at[idx_v.at[t]], m_bufs[b], gsem[b]
            ).wait()

        def write_start(t, b):
            blk = base_blk + t
            s = blk // cpr
            c = blk % cpr
            for eb in range(EMB // 8):
                tile = (s * (EMB // 8) + eb) * cpr + c
                pltpu.async_copy(
                    t_bufs[b].at[pl.ds(8 * eb, 8), pl.ds(0, CHUNK)],
                    out_hbm.at[tile],
                    osem[b],
                )

        def write_wait(t, b):
            blk = base_blk + t
            s = blk // cpr
            c = blk % cpr
            for eb in range(EMB // 8):
                tile = (s * (EMB // 8) + eb) * cpr + c
                pltpu.make_async_copy(
                    t_bufs[b].at[pl.ds(8 * eb, 8), pl.ds(0, CHUNK)],
                    out_hbm.at[tile],
                    osem[b],
                ).wait()

        evecs = [lanes + 16 * ec for ec in range(EMB // 16)]

        def transpose(b):
            m = m_bufs[b]
            mt = t_bufs[b]

            def jrow(j2, carry):
                jss = []
                vals = []
                for ju in range(4):
                    j = j2 * 4 + ju
                    jss.append(jnp.full((16,), 0, jnp.int32) + j)
                    for ec in range(EMB // 16):
                        vals.append(m[j, pl.ds(ec * 16, 16)])
                k = 0
                for ju in range(4):
                    for ec in range(EMB // 16):
                        plsc.store_scatter(mt, [evecs[ec], jss[ju]], vals[k])
                        k += 1
                return carry

            lax.fori_loop(0, CHUNK // 4, jrow, 0)

        for b in range(NBUF):
            gather_start(b, b)

        def outer(o, carry):
            for b in range(NBUF):
                t = o * NBUF + b

                gather_wait(t, b)

                @pl.when(o > 0)
                def _retire():
                    write_wait(t - NBUF, b)

                transpose(b)
                write_start(t, b)

                @pl.when(o < n_outer - 1)
                def _refill():
                    gather_start(t + NBUF, b)

            return carry

        lax.fori_loop(0, n_outer, outer, 0)

        for b in range(NBUF):
            write_wait((n_outer - 1) * NBUF + b, b)

    return k(idx2, table)


def kernel(indices, table):
    batch, seq = indices.shape
    idx2 = indices.T.reshape(seq * batch // CHUNK, CHUNK).astype(jnp.int32)
    table = jnp.pad(table, ((0, 0), (0, TPAD - EMB)))
    table = table.reshape(table.shape[0] * (TPAD // EMB), EMB)
    idx2 = idx2 * (TPAD // EMB)
    cpr = batch // CHUNK
    out_t = _sc_gather_t(idx2, table, batch)
    out6 = out_t.reshape(seq, EMB // 8, cpr, 8, CHUNK)
    return out6.transpose(2, 4, 0, 1, 3).reshape(batch, seq, EMB)
